# baseline (speedup 1.0000x reference)
import functools

import jax
import jax.numpy as jnp
from jax import lax
from jax.experimental import pallas as pl
from jax.experimental.pallas import tpu as pltpu
from jax.experimental.pallas import tpu_sc as plsc

D = 64
NC = 2
NS = 16
NW = NC * NS
SEQ = 200
SUBS = ((0, 128), (128, 72))
NBUF = 2


def _sc_gather(idx_flat, table, n_seqs):
    seqs_per_w = n_seqs // NW
    per_w = seqs_per_w * SEQ
    mesh = plsc.VectorSubcoreMesh(core_axis_name="c", subcore_axis_name="s")

    @functools.partial(
        pl.kernel,
        mesh=mesh,
        out_type=jax.ShapeDtypeStruct((n_seqs, SEQ, D), jnp.float32),
        compiler_params=pltpu.CompilerParams(use_tc_tiling_on_sc=False),
        scratch_types=[
            pltpu.VMEM((NBUF * SEQ,), jnp.int32),
            pltpu.VMEM((NBUF * SEQ, D), jnp.float32),
            pltpu.SemaphoreType.DMA,
            pltpu.SemaphoreType.DMA,
            pltpu.SemaphoreType.DMA,
            pltpu.SemaphoreType.DMA,
            pltpu.SemaphoreType.DMA,
            pltpu.SemaphoreType.DMA,
        ],
    )
    def k(idx_hbm, table_hbm, out_hbm, idx_v, rows_v, g0, g1, w0, w1, i0, i1):
        gsem = (g0, g1)
        wsem = (w0, w1)
        isem = (i0, i1)
        wid = lax.axis_index("s") * NC + lax.axis_index("c")
        base = wid * per_w
        seq_base = wid * seqs_per_w

        def idx_src(g):
            return idx_hbm.at[pl.ds(base + g * SEQ, SEQ)]

        def idx_dst(slot):
            return idx_v.at[pl.ds(slot * SEQ, SEQ)]

        def rows(slot):
            return rows_v.at[pl.ds(slot * SEQ, SEQ)]

        def out_dst(g):
            return out_hbm.at[seq_base + g]

        def gather_copies(slot):
            return [
                pltpu.make_async_copy(
                    table_hbm.at[idx_v.at[pl.ds(slot * SEQ + o, n)]],
                    rows_v.at[pl.ds(slot * SEQ + o, n)],
                    gsem[slot],
                )
                for o, n in SUBS
            ]

        pltpu.sync_copy(idx_src(0), idx_dst(0))
        for c in gather_copies(0):
            c.start()
        pltpu.async_copy(idx_src(1), idx_dst(1), isem[1])

        def stage(g, slot):
            nslot = 1 - slot

            @pl.when(g >= 1)
            def _():
                pltpu.make_async_copy(rows(nslot), out_dst(g - 1), wsem[nslot]).wait()

            @pl.when(g + 1 < seqs_per_w)
            def _():
                pltpu.make_async_copy(idx_src(g + 1), idx_dst(nslot), isem[nslot]).wait()
                for c in gather_copies(nslot):
                    c.start()

            for c in gather_copies(slot):
                c.wait()

            @pl.when(g + 2 < seqs_per_w)
            def _():
                pltpu.async_copy(idx_src(g + 2), idx_dst(slot), isem[slot])

            pltpu.async_copy(rows(slot), out_dst(g), wsem[slot])

        def outer(p, carry):
            stage(NBUF * p, 0)
            stage(NBUF * p + 1, 1)
            return carry

        lax.fori_loop(0, seqs_per_w // NBUF, outer, 0)

        last = seqs_per_w - 1
        pltpu.make_async_copy(rows(last % NBUF), out_dst(last), wsem[last % NBUF]).wait()

    return k(idx_flat, table)


def kernel(x, table):
    b, s = x.shape
    idx_flat = x.reshape(b * s).astype(jnp.int32)
    return _sc_gather(idx_flat, table, b)
